# baseline (speedup 1.0000x reference)
import jax
import jax.numpy as jnp
from jax.experimental import pallas as pl
from jax.experimental.pallas import tpu as pltpu

_BB = 512


def _pe_add_kernel(x_ref, pe_ref, o_ref):
    L = x_ref.shape[1]
    o_ref[...] = x_ref[...] + pe_ref[:L, :][None, :, :]


def kernel(x, pe):
    B, L, D = x.shape
    grid = (B // _BB,)
    return pl.pallas_call(
        _pe_add_kernel,
        grid=grid,
        in_specs=[
            pl.BlockSpec((_BB, L, D), lambda i: (i, 0, 0)),
            pl.BlockSpec(pe.shape, lambda i: (0, 0)),
        ],
        out_specs=pl.BlockSpec((_BB, L, D), lambda i: (i, 0, 0)),
        out_shape=jax.ShapeDtypeStruct((B, L, D), x.dtype),
        compiler_params=pltpu.CompilerParams(
            dimension_semantics=("parallel",),
            vmem_limit_bytes=100 * 1024 * 1024,
        ),
    )(x, pe)


# speedup vs baseline: 1.0339x; 1.0339x over previous
import jax
import jax.numpy as jnp
from jax.experimental import pallas as pl
from jax.experimental.pallas import tpu as pltpu

_C = 128
_NBUF = 4


def _pe_add_kernel(x_ref, pe_ref, o_ref, ibuf, obuf, sem_in, sem_out):
    B, L, D = x_ref.shape
    nchunk = B // _C
    pos = pe_ref[:L, :][None, :, :]

    def in_cp(i):
        s = i % _NBUF
        return pltpu.make_async_copy(
            x_ref.at[pl.ds(i * _C, _C)], ibuf.at[s], sem_in.at[s])

    def out_cp(i):
        s = i % _NBUF
        return pltpu.make_async_copy(
            obuf.at[s], o_ref.at[pl.ds(i * _C, _C)], sem_out.at[s])

    for i in range(min(_NBUF, nchunk)):
        in_cp(i).start()
    for i in range(nchunk):
        s = i % _NBUF
        in_cp(i).wait()
        if i >= _NBUF:
            out_cp(i - _NBUF).wait()
        obuf[s] = ibuf[s] + pos
        out_cp(i).start()
        if i + _NBUF < nchunk:
            in_cp(i + _NBUF).start()
    for i in range(max(0, nchunk - _NBUF), nchunk):
        out_cp(i).wait()


def kernel(x, pe):
    B, L, D = x.shape
    return pl.pallas_call(
        _pe_add_kernel,
        in_specs=[
            pl.BlockSpec(memory_space=pltpu.MemorySpace.HBM),
            pl.BlockSpec(memory_space=pltpu.MemorySpace.VMEM),
        ],
        out_specs=pl.BlockSpec(memory_space=pltpu.MemorySpace.HBM),
        out_shape=jax.ShapeDtypeStruct((B, L, D), x.dtype),
        scratch_shapes=[
            pltpu.VMEM((_NBUF, _C, L, D), x.dtype),
            pltpu.VMEM((_NBUF, _C, L, D), x.dtype),
            pltpu.SemaphoreType.DMA((_NBUF,)),
            pltpu.SemaphoreType.DMA((_NBUF,)),
        ],
        compiler_params=pltpu.CompilerParams(
            vmem_limit_bytes=100 * 1024 * 1024,
        ),
    )(x, pe)
